# baseline (speedup 1.0000x reference)
import functools

import numpy as np
import jax
import jax.numpy as jnp
from jax import lax
from jax.experimental import pallas as pl
from jax.experimental.pallas import tpu as pltpu
from jax.experimental.pallas import tpu_sc as plsc


def _bf(x):
    return x.astype(jnp.bfloat16)


def _prep_kernel(x_ref, n1_ref, wq_ref, wk_ref, wv_ref, q_ref, k_ref, v_ref):
    x = x_ref[...]
    nx = x * lax.rsqrt(jnp.mean(x * x, axis=-1, keepdims=True) + 1e-6)
    nx = nx * n1_ref[...]
    nxb = _bf(nx)
    q_ref[...] = jnp.dot(nxb, _bf(wq_ref[...]), preferred_element_type=jnp.float32)
    k_ref[...] = jnp.dot(nxb, _bf(wk_ref[...]), preferred_element_type=jnp.float32)
    v_ref[...] = jnp.dot(nxb, _bf(wv_ref[...]), preferred_element_type=jnp.float32)


def _attn_kernel(q_ref, k_ref, v_ref, x_ref, wo_ref, o_ref, *, dk, tb):
    h = pl.program_id(0)
    t = pl.program_id(1)
    q = _bf(q_ref[...])
    kc = _bf(k_ref[...])
    s = lax.dot_general(q, kc, (((1,), (1,)), ((), ())),
                        preferred_element_type=jnp.float32)
    s = s * (1.0 / np.sqrt(dk))
    p = jnp.exp(s)
    l = jnp.maximum(jnp.sum(p, axis=-1, keepdims=True), 1e-30)
    o = jnp.dot(_bf(p), _bf(v_ref[...]), preferred_element_type=jnp.float32) / l
    contrib = jnp.dot(_bf(o), _bf(wo_ref[...]), preferred_element_type=jnp.float32)
    rows = pl.ds(t * tb, tb)

    @pl.when(h == 0)
    def _():
        o_ref[rows, :] = x_ref[...] + contrib

    @pl.when(h > 0)
    def _():
        o_ref[rows, :] += contrib


def _router_kernel(x1_ref, n2_ref, rw_ref, bias_ref,
                   nx2_ref, p0_ref, p1_ref, w1_ref, w2_ref, be_ref, bv_ref,
                   *, ne, blk, nb):
    x = x1_ref[...]
    nx = x * lax.rsqrt(jnp.mean(x * x, axis=-1, keepdims=True) + 1e-6)
    nx = nx * n2_ref[...]
    nx2_ref[...] = nx
    logits = jnp.dot(nx, rw_ref[...], preferred_element_type=jnp.float32,
                     precision=jax.lax.Precision.HIGHEST) + bias_ref[...]
    lm = jnp.max(logits, axis=-1, keepdims=True)
    el = jnp.exp(logits - lm)
    rw = el / jnp.sum(el, axis=-1, keepdims=True)
    t = rw.shape[0]
    i32 = jnp.int32
    iota = lax.broadcasted_iota(i32, (t, ne), 1)
    m1 = jnp.max(rw, axis=-1, keepdims=True)
    i1 = jnp.min(jnp.where(rw == m1, iota, ne), axis=-1, keepdims=True)
    mask1 = iota == i1
    rw2 = jnp.where(mask1, -jnp.inf, rw)
    m2 = jnp.max(rw2, axis=-1, keepdims=True)
    i2 = jnp.min(jnp.where(rw2 == m2, iota, ne), axis=-1, keepdims=True)
    mask2 = iota == i2
    e2 = jnp.exp(m2 - m1)
    w1_ref[...] = 1.0 / (1.0 + e2)
    w2_ref[...] = e2 / (1.0 + e2)
    cnt = mask1.astype(i32) + mask2.astype(i32)
    c = cnt
    s = 1
    while s < t:
        sh = jnp.concatenate([jnp.zeros((s, ne), i32), c[:t - s, :]], axis=0)
        c = c + sh
        s *= 2
    excl = c - cnt
    counts = c[t - 1:t, :]
    padded = ((counts + blk - 1) // blk) * blk
    cp = padded
    s = 1
    while s < ne:
        shl = jnp.concatenate(
            [jnp.zeros((1, s), i32), cp[:, :ne - s]], axis=1)
        cp = cp + shl
        s *= 2
    offs = cp - padded
    pos = excl + offs
    p0_ref[...] = jnp.sum(jnp.where(mask1, pos, 0), axis=1, keepdims=True)
    p1_ref[...] = jnp.sum(jnp.where(mask2, pos, 0), axis=1, keepdims=True)
    bs = lax.broadcasted_iota(i32, (nb, ne), 0) * blk
    be = jnp.sum((bs >= cp).astype(i32), axis=1, keepdims=True)
    be_ref[...] = jnp.minimum(be, ne - 1)
    total = jnp.sum(jnp.where(
        lax.broadcasted_iota(i32, (nb, ne), 1) == ne - 1, cp, 0),
        axis=1, keepdims=True)
    bv_ref[...] = (bs[:, :1] < total).astype(i32)


def _shared_kernel(x1_ref, nx2_ref, wg_ref, wu_ref, wd_ref, o_ref):
    x = _bf(nx2_ref[...])
    g = jnp.dot(x, _bf(wg_ref[...]), preferred_element_type=jnp.float32)
    u = jnp.dot(x, _bf(wu_ref[...]), preferred_element_type=jnp.float32)
    hdn = jax.nn.silu(g) * u
    o_ref[...] = x1_ref[...] + jnp.dot(_bf(hdn), _bf(wd_ref[...]),
                                       preferred_element_type=jnp.float32)


def _group_kernel(be_ref, bv_ref, y_ref, wg_ref, wu_ref, wd_ref, z_ref):
    b = pl.program_id(0)
    j = pl.program_id(1)

    @pl.when(bv_ref[b] != 0)
    def _():
        x = _bf(y_ref[...])
        g = jnp.dot(x, _bf(wg_ref[0]), preferred_element_type=jnp.float32)
        u = jnp.dot(x, _bf(wu_ref[0]), preferred_element_type=jnp.float32)
        hdn = jax.nn.silu(g) * u
        z = jnp.dot(_bf(hdn), _bf(wd_ref[0]), preferred_element_type=jnp.float32)

        @pl.when(j == 0)
        def _():
            z_ref[...] = z

        @pl.when(j > 0)
        def _():
            z_ref[...] += z


def _make_scatter_dispatch(T, P, d, nw, ch):
    tok_per_w = T // nw
    nch = tok_per_w // ch
    mesh = plsc.VectorSubcoreMesh(core_axis_name="c", subcore_axis_name="s")

    @functools.partial(
        pl.kernel, mesh=mesh,
        out_type=jax.ShapeDtypeStruct((P, d), jnp.float32),
        scratch_types=[pltpu.VMEM((nch, ch), jnp.int32),
                       pltpu.VMEM((nch, ch), jnp.int32)]
                      + [pltpu.VMEM((ch, d), jnp.float32)] * 2
                      + [pltpu.SemaphoreType.DMA, pltpu.SemaphoreType.DMA])
    def dispatch(x_hbm, p0_hbm, p1_hbm, y_hbm, i0_v, i1_v, b0, b1, gsem, wsem):
        wid = lax.axis_index("s") * 2 + lax.axis_index("c")
        base = wid * tok_per_w
        pltpu.sync_copy(p0_hbm.at[wid], i0_v)
        pltpu.sync_copy(p1_hbm.at[wid], i1_v)
        bufs = [b0, b1]
        g = [None] * nch
        w = [None] * nch
        nbuf = min(2, nch)
        for c in range(nbuf):
            off = pl.multiple_of(base + c * ch, 8)
            g[c] = pltpu.async_copy(x_hbm.at[pl.ds(off, ch)], bufs[c % 2], gsem)
        for c in range(nch):
            g[c].wait()
            w0 = pltpu.async_copy(bufs[c % 2], y_hbm.at[i0_v.at[c]], wsem)
            w1 = pltpu.async_copy(bufs[c % 2], y_hbm.at[i1_v.at[c]], wsem)
            w[c] = (w0, w1)
            if c + nbuf < nch:
                w[c][0].wait()
                w[c][1].wait()
                off = pl.multiple_of(base + (c + nbuf) * ch, 8)
                g[c + nbuf] = pltpu.async_copy(x_hbm.at[pl.ds(off, ch)],
                                               bufs[(c + nbuf) % 2], gsem)
        for c in range(max(0, nch - nbuf), nch):
            w[c][0].wait()
            w[c][1].wait()

    return dispatch


def _make_dispatch(P, d, nw, ch, dtype=jnp.float32):
    rows_per_w = P // nw
    nch = rows_per_w // ch
    nbuf = min(3, nch)
    mesh = plsc.VectorSubcoreMesh(core_axis_name="c", subcore_axis_name="s")

    @functools.partial(
        pl.kernel, mesh=mesh,
        out_type=jax.ShapeDtypeStruct((P, d), dtype),
        scratch_types=[pltpu.VMEM((rows_per_w,), jnp.int32)]
                      + [pltpu.VMEM((ch, d), dtype)] * 3
                      + [pltpu.SemaphoreType.DMA, pltpu.SemaphoreType.DMA])
    def dispatch(x_hbm, idx_hbm, y_hbm, idx_v, b0, b1, b2, gsem, wsem):
        wid = lax.axis_index("s") * 2 + lax.axis_index("c")
        base = wid * rows_per_w
        pltpu.sync_copy(idx_hbm.at[pl.ds(pl.multiple_of(base, 8), rows_per_w)],
                        idx_v)
        bufs = [b0, b1, b2]
        g = [None] * nch
        w = [None] * nch
        for i in range(nbuf):
            g[i] = pltpu.async_copy(x_hbm.at[idx_v.at[pl.ds(i * ch, ch)]],
                                    bufs[i % 3], gsem)
        for i in range(nch):
            g[i].wait()
            off = pl.multiple_of(base + i * ch, 8)
            w[i] = pltpu.async_copy(bufs[i % 3], y_hbm.at[pl.ds(off, ch)], wsem)
            if i + nbuf < nch:
                w[i].wait()
                g[i + nbuf] = pltpu.async_copy(
                    x_hbm.at[idx_v.at[pl.ds((i + nbuf) * ch, ch)]],
                    bufs[(i + nbuf) % 3], gsem)
        for i in range(max(0, nch - nbuf), nch):
            w[i].wait()

    return dispatch


def _make_gather2(T, d, nw, ch, dtype):
    rows_per_w = T // nw
    nch = rows_per_w // ch
    mesh = plsc.VectorSubcoreMesh(core_axis_name="c", subcore_axis_name="s")

    @functools.partial(
        pl.kernel, mesh=mesh,
        out_type=(jax.ShapeDtypeStruct((T, d), dtype),
                  jax.ShapeDtypeStruct((T, d), dtype)),
        scratch_types=[pltpu.VMEM((rows_per_w,), jnp.int32),
                       pltpu.VMEM((rows_per_w,), jnp.int32)]
                      + [pltpu.VMEM((ch, d), dtype)] * 4
                      + [pltpu.SemaphoreType.DMA, pltpu.SemaphoreType.DMA])
    def gather2(z_hbm, p0_hbm, p1_hbm, z0_hbm, z1_hbm,
                i0_v, i1_v, a0, a1, b0, b1, gsem, wsem):
        wid = lax.axis_index("s") * 2 + lax.axis_index("c")
        base = wid * rows_per_w
        boff = pl.multiple_of(base, 8)
        pltpu.sync_copy(p0_hbm.at[pl.ds(boff, rows_per_w)], i0_v)
        pltpu.sync_copy(p1_hbm.at[pl.ds(boff, rows_per_w)], i1_v)
        bufs0 = [a0, b0]
        bufs1 = [a1, b1]
        g = [None] * nch
        w = [None] * nch
        nbuf = min(2, nch)

        def fire(i):
            s0 = pltpu.async_copy(z_hbm.at[i0_v.at[pl.ds(i * ch, ch)]],
                                  bufs0[i % 2], gsem)
            s1 = pltpu.async_copy(z_hbm.at[i1_v.at[pl.ds(i * ch, ch)]],
                                  bufs1[i % 2], gsem)
            return (s0, s1)

        for i in range(nbuf):
            g[i] = fire(i)
        for i in range(nch):
            g[i][0].wait()
            g[i][1].wait()
            off = pl.multiple_of(base + i * ch, 8)
            w0 = pltpu.async_copy(bufs0[i % 2], z0_hbm.at[pl.ds(off, ch)], wsem)
            w1 = pltpu.async_copy(bufs1[i % 2], z1_hbm.at[pl.ds(off, ch)], wsem)
            w[i] = (w0, w1)
            if i + nbuf < nch:
                w[i][0].wait()
                w[i][1].wait()
                g[i + nbuf] = fire(i + nbuf)
        for i in range(max(0, nch - nbuf), nch):
            w[i][0].wait()
            w[i][1].wait()

    return gather2


def _combine_add_kernel(acc_ref, z0_ref, z1_ref, w1_ref, w2_ref, o_ref):
    o_ref[...] = (acc_ref[...]
                  + w1_ref[...] * z0_ref[...].astype(jnp.float32)
                  + w2_ref[...] * z1_ref[...].astype(jnp.float32))


def kernel(X, norm1_w, norm2_w, W_q, W_k, W_v, W_o, router_W, expert_bias,
           sh_wg, sh_wu, sh_wd, ex_wg, ex_wu, ex_wd):
    b, s, d = X.shape
    ne, _, dff = ex_wg.shape
    h = 4
    dk = d // h
    dkv = d // 4
    f32 = jnp.float32
    i32 = jnp.int32

    Xf = X.reshape(b * s, d)
    n1 = norm1_w.reshape(1, d)
    n2 = norm2_w.reshape(1, d)
    bias = expert_bias.reshape(1, ne)
    T = b * s
    TB = 512
    nt = T // TB
    topk = 2
    NPAIR = T * topk
    BLK = 512
    NB = NPAIR // BLK + ne
    P = NB * BLK

    q, kc, vc = pl.pallas_call(
        _prep_kernel,
        out_shape=(jax.ShapeDtypeStruct((T, d), f32),
                   jax.ShapeDtypeStruct((T, dkv), f32),
                   jax.ShapeDtypeStruct((T, dkv), f32)),
    )(Xf, n1, W_q, W_k, W_v)

    ATB = 1024
    nat = T // ATB
    x1 = pl.pallas_call(
        functools.partial(_attn_kernel, dk=dk, tb=ATB),
        grid=(h, nat),
        in_specs=[
            pl.BlockSpec((ATB, dk), lambda i, t: (t, i)),
            pl.BlockSpec((T, dkv), lambda i, t: (0, 0)),
            pl.BlockSpec((T, dkv), lambda i, t: (0, 0)),
            pl.BlockSpec((ATB, d), lambda i, t: (t, 0)),
            pl.BlockSpec((dkv, d), lambda i, t: (i, 0)),
        ],
        out_specs=pl.BlockSpec((T, d), lambda i, t: (0, 0)),
        out_shape=jax.ShapeDtypeStruct((T, d), f32),
        compiler_params=pltpu.CompilerParams(
            dimension_semantics=("arbitrary", "arbitrary")),
    )(q, kc, vc, Xf, W_o)

    nx2, p0c, p1c, w1, w2, bec, bvc = pl.pallas_call(
        functools.partial(_router_kernel, ne=ne, blk=BLK, nb=NB),
        out_shape=(jax.ShapeDtypeStruct((T, d), f32),
                   jax.ShapeDtypeStruct((T, 1), i32),
                   jax.ShapeDtypeStruct((T, 1), i32),
                   jax.ShapeDtypeStruct((T, 1), f32),
                   jax.ShapeDtypeStruct((T, 1), f32),
                   jax.ShapeDtypeStruct((NB, 1), i32),
                   jax.ShapeDtypeStruct((NB, 1), i32)),
    )(x1, n2, router_W, bias)
    p0 = p0c[:, 0]
    p1 = p1c[:, 0]
    block_expert = bec[:, 0]
    block_valid = bvc[:, 0]

    nw = 32
    dch = 32
    dnch = T // nw // dch
    y = _make_scatter_dispatch(T, P, d, nw, dch)(
        nx2, p0.reshape(nw, dnch, dch), p1.reshape(nw, dnch, dch))

    acc = pl.pallas_call(
        _shared_kernel,
        grid=(nt,),
        in_specs=[
            pl.BlockSpec((TB, d), lambda t: (t, 0)),
            pl.BlockSpec((TB, d), lambda t: (t, 0)),
            pl.BlockSpec((d, dff), lambda t: (0, 0)),
            pl.BlockSpec((d, dff), lambda t: (0, 0)),
            pl.BlockSpec((dff, d), lambda t: (0, 0)),
        ],
        out_specs=pl.BlockSpec((TB, d), lambda t: (t, 0)),
        out_shape=jax.ShapeDtypeStruct((T, d), f32),
    )(x1, nx2, sh_wg, sh_wu, sh_wd)

    z = pl.pallas_call(
        _group_kernel,
        grid_spec=pltpu.PrefetchScalarGridSpec(
            num_scalar_prefetch=2,
            grid=(NB, 2),
            in_specs=[
                pl.BlockSpec((BLK, d), lambda bb, j, be, bv: (bb, 0)),
                pl.BlockSpec((1, d, dff // 2),
                             lambda bb, j, be, bv: (be[bb], 0, j)),
                pl.BlockSpec((1, d, dff // 2),
                             lambda bb, j, be, bv: (be[bb], 0, j)),
                pl.BlockSpec((1, dff // 2, d),
                             lambda bb, j, be, bv: (be[bb], j, 0)),
            ],
            out_specs=pl.BlockSpec((BLK, d), lambda bb, j, be, bv: (bb, 0)),
        ),
        out_shape=jax.ShapeDtypeStruct((P, d), f32),
        compiler_params=pltpu.CompilerParams(
            dimension_semantics=("arbitrary", "arbitrary")),
    )(block_expert, block_valid, y, ex_wg, ex_wu, ex_wd)

    z0, z1 = _make_gather2(T, d, nw, 16, f32)(z, p0, p1)
    out = pl.pallas_call(
        _combine_add_kernel,
        grid=(nt,),
        in_specs=[
            pl.BlockSpec((TB, d), lambda t: (t, 0)),
            pl.BlockSpec((TB, d), lambda t: (t, 0)),
            pl.BlockSpec((TB, d), lambda t: (t, 0)),
            pl.BlockSpec((TB, 1), lambda t: (t, 0)),
            pl.BlockSpec((TB, 1), lambda t: (t, 0)),
        ],
        out_specs=pl.BlockSpec((TB, d), lambda t: (t, 0)),
        out_shape=jax.ShapeDtypeStruct((T, d), f32),
    )(acc, z0, z1, w1, w2)

    return out.reshape(b, s, d)


# speedup vs baseline: 1.1035x; 1.1035x over previous
import functools

import numpy as np
import jax
import jax.numpy as jnp
from jax import lax
from jax.experimental import pallas as pl
from jax.experimental.pallas import tpu as pltpu
from jax.experimental.pallas import tpu_sc as plsc


def _bf(x):
    return x.astype(jnp.bfloat16)


def _prep_kernel(x_ref, n1_ref, wq_ref, wk_ref, wv_ref, q_ref, k_ref, v_ref):
    x = x_ref[...]
    nx = x * lax.rsqrt(jnp.mean(x * x, axis=-1, keepdims=True) + 1e-6)
    nx = nx * n1_ref[...]
    nxb = _bf(nx)
    q_ref[...] = jnp.dot(nxb, _bf(wq_ref[...]), preferred_element_type=jnp.float32)
    k_ref[...] = jnp.dot(nxb, _bf(wk_ref[...]), preferred_element_type=jnp.float32)
    v_ref[...] = jnp.dot(nxb, _bf(wv_ref[...]), preferred_element_type=jnp.float32)


def _attn_kernel(q_ref, k_ref, v_ref, x_ref, wo_ref, o_ref, *, dk, tb):
    h = pl.program_id(0)
    t = pl.program_id(1)
    q = _bf(q_ref[...])
    kc = _bf(k_ref[...])
    s = lax.dot_general(q, kc, (((1,), (1,)), ((), ())),
                        preferred_element_type=jnp.float32)
    s = s * (1.0 / np.sqrt(dk))
    p = jnp.exp(s)
    l = jnp.maximum(jnp.sum(p, axis=-1, keepdims=True), 1e-30)
    o = jnp.dot(_bf(p), _bf(v_ref[...]), preferred_element_type=jnp.float32) / l
    contrib = jnp.dot(_bf(o), _bf(wo_ref[...]), preferred_element_type=jnp.float32)
    rows = pl.ds(t * tb, tb)

    @pl.when(h == 0)
    def _():
        o_ref[rows, :] = x_ref[...] + contrib

    @pl.when(h > 0)
    def _():
        o_ref[rows, :] += contrib


def _router_kernel(x1_ref, n2_ref, rw_ref, bias_ref,
                   nx2_ref, p0_ref, p1_ref, w1_ref, w2_ref, be_ref, bv_ref,
                   *, ne, blk, nb):
    x = x1_ref[...]
    nx = x * lax.rsqrt(jnp.mean(x * x, axis=-1, keepdims=True) + 1e-6)
    nx = nx * n2_ref[...]
    nx2_ref[...] = nx
    logits = jnp.dot(nx, rw_ref[...], preferred_element_type=jnp.float32,
                     precision=jax.lax.Precision.HIGHEST) + bias_ref[...]
    lm = jnp.max(logits, axis=-1, keepdims=True)
    el = jnp.exp(logits - lm)
    rw = el / jnp.sum(el, axis=-1, keepdims=True)
    t = rw.shape[0]
    i32 = jnp.int32
    iota = lax.broadcasted_iota(i32, (t, ne), 1)
    m1 = jnp.max(rw, axis=-1, keepdims=True)
    i1 = jnp.min(jnp.where(rw == m1, iota, ne), axis=-1, keepdims=True)
    mask1 = iota == i1
    rw2 = jnp.where(mask1, -jnp.inf, rw)
    m2 = jnp.max(rw2, axis=-1, keepdims=True)
    i2 = jnp.min(jnp.where(rw2 == m2, iota, ne), axis=-1, keepdims=True)
    mask2 = iota == i2
    e2 = jnp.exp(m2 - m1)
    w1_ref[...] = 1.0 / (1.0 + e2)
    w2_ref[...] = e2 / (1.0 + e2)
    cnt = mask1.astype(i32) + mask2.astype(i32)
    c = cnt
    s = 1
    while s < t:
        sh = jnp.concatenate([jnp.zeros((s, ne), i32), c[:t - s, :]], axis=0)
        c = c + sh
        s *= 2
    excl = c - cnt
    counts = c[t - 1:t, :]
    padded = ((counts + blk - 1) // blk) * blk
    cp = padded
    s = 1
    while s < ne:
        shl = jnp.concatenate(
            [jnp.zeros((1, s), i32), cp[:, :ne - s]], axis=1)
        cp = cp + shl
        s *= 2
    offs = cp - padded
    pos = excl + offs
    p0_ref[...] = jnp.sum(jnp.where(mask1, pos, 0), axis=1, keepdims=True)
    p1_ref[...] = jnp.sum(jnp.where(mask2, pos, 0), axis=1, keepdims=True)
    bs = lax.broadcasted_iota(i32, (nb, ne), 0) * blk
    be = jnp.sum((bs >= cp).astype(i32), axis=1, keepdims=True)
    be_ref[...] = jnp.minimum(be, ne - 1)
    total = jnp.sum(jnp.where(
        lax.broadcasted_iota(i32, (nb, ne), 1) == ne - 1, cp, 0),
        axis=1, keepdims=True)
    bv_ref[...] = (bs[:, :1] < total).astype(i32)


def _shared_kernel(x1_ref, nx2_ref, wg_ref, wu_ref, wd_ref, o_ref):
    x = _bf(nx2_ref[...])
    g = jnp.dot(x, _bf(wg_ref[...]), preferred_element_type=jnp.float32)
    u = jnp.dot(x, _bf(wu_ref[...]), preferred_element_type=jnp.float32)
    hdn = jax.nn.silu(g) * u
    o_ref[...] = x1_ref[...] + jnp.dot(_bf(hdn), _bf(wd_ref[...]),
                                       preferred_element_type=jnp.float32)


def _group_kernel(be_ref, bv_ref, y_ref, wg_ref, wu_ref, wd_ref, z_ref):
    b = pl.program_id(0)

    @pl.when(bv_ref[b] != 0)
    def _():
        x = _bf(y_ref[...])
        g = jnp.dot(x, _bf(wg_ref[0]), preferred_element_type=jnp.float32)
        u = jnp.dot(x, _bf(wu_ref[0]), preferred_element_type=jnp.float32)
        hdn = jax.nn.silu(g) * u
        z = jnp.dot(_bf(hdn), _bf(wd_ref[0]), preferred_element_type=jnp.float32)
        z_ref[...] = z


def _make_scatter_dispatch(T, P, d, nw, ch):
    tok_per_w = T // nw
    nch = tok_per_w // ch
    mesh = plsc.VectorSubcoreMesh(core_axis_name="c", subcore_axis_name="s")

    @functools.partial(
        pl.kernel, mesh=mesh,
        out_type=jax.ShapeDtypeStruct((P, d), jnp.float32),
        scratch_types=[pltpu.VMEM((nch, ch), jnp.int32),
                       pltpu.VMEM((nch, ch), jnp.int32)]
                      + [pltpu.VMEM((ch, d), jnp.float32)] * 2
                      + [pltpu.SemaphoreType.DMA, pltpu.SemaphoreType.DMA])
    def dispatch(x_hbm, p0_hbm, p1_hbm, y_hbm, i0_v, i1_v, b0, b1, gsem, wsem):
        wid = lax.axis_index("s") * 2 + lax.axis_index("c")
        base = wid * tok_per_w
        pltpu.sync_copy(p0_hbm.at[wid], i0_v)
        pltpu.sync_copy(p1_hbm.at[wid], i1_v)
        bufs = [b0, b1]
        g = [None] * nch
        w = [None] * nch
        nbuf = min(2, nch)
        for c in range(nbuf):
            off = pl.multiple_of(base + c * ch, 8)
            g[c] = pltpu.async_copy(x_hbm.at[pl.ds(off, ch)], bufs[c % 2], gsem)
        for c in range(nch):
            g[c].wait()
            w0 = pltpu.async_copy(bufs[c % 2], y_hbm.at[i0_v.at[c]], wsem)
            w1 = pltpu.async_copy(bufs[c % 2], y_hbm.at[i1_v.at[c]], wsem)
            w[c] = (w0, w1)
            if c + nbuf < nch:
                w[c][0].wait()
                w[c][1].wait()
                off = pl.multiple_of(base + (c + nbuf) * ch, 8)
                g[c + nbuf] = pltpu.async_copy(x_hbm.at[pl.ds(off, ch)],
                                               bufs[(c + nbuf) % 2], gsem)
        for c in range(max(0, nch - nbuf), nch):
            w[c][0].wait()
            w[c][1].wait()

    return dispatch


def _make_dispatch(P, d, nw, ch, dtype=jnp.float32):
    rows_per_w = P // nw
    nch = rows_per_w // ch
    nbuf = min(3, nch)
    mesh = plsc.VectorSubcoreMesh(core_axis_name="c", subcore_axis_name="s")

    @functools.partial(
        pl.kernel, mesh=mesh,
        out_type=jax.ShapeDtypeStruct((P, d), dtype),
        scratch_types=[pltpu.VMEM((rows_per_w,), jnp.int32)]
                      + [pltpu.VMEM((ch, d), dtype)] * 3
                      + [pltpu.SemaphoreType.DMA, pltpu.SemaphoreType.DMA])
    def dispatch(x_hbm, idx_hbm, y_hbm, idx_v, b0, b1, b2, gsem, wsem):
        wid = lax.axis_index("s") * 2 + lax.axis_index("c")
        base = wid * rows_per_w
        pltpu.sync_copy(idx_hbm.at[pl.ds(pl.multiple_of(base, 8), rows_per_w)],
                        idx_v)
        bufs = [b0, b1, b2]
        g = [None] * nch
        w = [None] * nch
        for i in range(nbuf):
            g[i] = pltpu.async_copy(x_hbm.at[idx_v.at[pl.ds(i * ch, ch)]],
                                    bufs[i % 3], gsem)
        for i in range(nch):
            g[i].wait()
            off = pl.multiple_of(base + i * ch, 8)
            w[i] = pltpu.async_copy(bufs[i % 3], y_hbm.at[pl.ds(off, ch)], wsem)
            if i + nbuf < nch:
                w[i].wait()
                g[i + nbuf] = pltpu.async_copy(
                    x_hbm.at[idx_v.at[pl.ds((i + nbuf) * ch, ch)]],
                    bufs[(i + nbuf) % 3], gsem)
        for i in range(max(0, nch - nbuf), nch):
            w[i].wait()

    return dispatch


def _make_gather2(T, d, nw, ch, dtype):
    rows_per_w = T // nw
    nch = rows_per_w // ch
    mesh = plsc.VectorSubcoreMesh(core_axis_name="c", subcore_axis_name="s")

    @functools.partial(
        pl.kernel, mesh=mesh,
        out_type=(jax.ShapeDtypeStruct((T, d), dtype),
                  jax.ShapeDtypeStruct((T, d), dtype)),
        scratch_types=[pltpu.VMEM((rows_per_w,), jnp.int32),
                       pltpu.VMEM((rows_per_w,), jnp.int32)]
                      + [pltpu.VMEM((ch, d), dtype)] * 4
                      + [pltpu.SemaphoreType.DMA, pltpu.SemaphoreType.DMA])
    def gather2(z_hbm, p0_hbm, p1_hbm, z0_hbm, z1_hbm,
                i0_v, i1_v, a0, a1, b0, b1, gsem, wsem):
        wid = lax.axis_index("s") * 2 + lax.axis_index("c")
        base = wid * rows_per_w
        boff = pl.multiple_of(base, 8)
        pltpu.sync_copy(p0_hbm.at[pl.ds(boff, rows_per_w)], i0_v)
        pltpu.sync_copy(p1_hbm.at[pl.ds(boff, rows_per_w)], i1_v)
        bufs0 = [a0, b0]
        bufs1 = [a1, b1]
        g = [None] * nch
        w = [None] * nch
        nbuf = min(2, nch)

        def fire(i):
            s0 = pltpu.async_copy(z_hbm.at[i0_v.at[pl.ds(i * ch, ch)]],
                                  bufs0[i % 2], gsem)
            s1 = pltpu.async_copy(z_hbm.at[i1_v.at[pl.ds(i * ch, ch)]],
                                  bufs1[i % 2], gsem)
            return (s0, s1)

        for i in range(nbuf):
            g[i] = fire(i)
        for i in range(nch):
            g[i][0].wait()
            g[i][1].wait()
            off = pl.multiple_of(base + i * ch, 8)
            w0 = pltpu.async_copy(bufs0[i % 2], z0_hbm.at[pl.ds(off, ch)], wsem)
            w1 = pltpu.async_copy(bufs1[i % 2], z1_hbm.at[pl.ds(off, ch)], wsem)
            w[i] = (w0, w1)
            if i + nbuf < nch:
                w[i][0].wait()
                w[i][1].wait()
                g[i + nbuf] = fire(i + nbuf)
        for i in range(max(0, nch - nbuf), nch):
            w[i][0].wait()
            w[i][1].wait()

    return gather2


def _combine_add_kernel(acc_ref, z0_ref, z1_ref, w1_ref, w2_ref, o_ref):
    o_ref[...] = (acc_ref[...]
                  + w1_ref[...] * z0_ref[...].astype(jnp.float32)
                  + w2_ref[...] * z1_ref[...].astype(jnp.float32))


def kernel(X, norm1_w, norm2_w, W_q, W_k, W_v, W_o, router_W, expert_bias,
           sh_wg, sh_wu, sh_wd, ex_wg, ex_wu, ex_wd):
    b, s, d = X.shape
    ne, _, dff = ex_wg.shape
    h = 4
    dk = d // h
    dkv = d // 4
    f32 = jnp.float32
    i32 = jnp.int32

    Xf = X.reshape(b * s, d)
    n1 = norm1_w.reshape(1, d)
    n2 = norm2_w.reshape(1, d)
    bias = expert_bias.reshape(1, ne)
    T = b * s
    TB = 512
    nt = T // TB
    topk = 2
    NPAIR = T * topk
    BLK = 512
    NB = NPAIR // BLK + ne
    P = NB * BLK

    q, kc, vc = pl.pallas_call(
        _prep_kernel,
        out_shape=(jax.ShapeDtypeStruct((T, d), f32),
                   jax.ShapeDtypeStruct((T, dkv), f32),
                   jax.ShapeDtypeStruct((T, dkv), f32)),
    )(Xf, n1, W_q, W_k, W_v)

    ATB = 1024
    nat = T // ATB
    x1 = pl.pallas_call(
        functools.partial(_attn_kernel, dk=dk, tb=ATB),
        grid=(h, nat),
        in_specs=[
            pl.BlockSpec((ATB, dk), lambda i, t: (t, i)),
            pl.BlockSpec((T, dkv), lambda i, t: (0, 0)),
            pl.BlockSpec((T, dkv), lambda i, t: (0, 0)),
            pl.BlockSpec((ATB, d), lambda i, t: (t, 0)),
            pl.BlockSpec((dkv, d), lambda i, t: (i, 0)),
        ],
        out_specs=pl.BlockSpec((T, d), lambda i, t: (0, 0)),
        out_shape=jax.ShapeDtypeStruct((T, d), f32),
        compiler_params=pltpu.CompilerParams(
            dimension_semantics=("arbitrary", "arbitrary")),
    )(q, kc, vc, Xf, W_o)

    nx2, p0c, p1c, w1, w2, bec, bvc = pl.pallas_call(
        functools.partial(_router_kernel, ne=ne, blk=BLK, nb=NB),
        out_shape=(jax.ShapeDtypeStruct((T, d), f32),
                   jax.ShapeDtypeStruct((T, 1), i32),
                   jax.ShapeDtypeStruct((T, 1), i32),
                   jax.ShapeDtypeStruct((T, 1), f32),
                   jax.ShapeDtypeStruct((T, 1), f32),
                   jax.ShapeDtypeStruct((NB, 1), i32),
                   jax.ShapeDtypeStruct((NB, 1), i32)),
    )(x1, n2, router_W, bias)
    p0 = p0c[:, 0]
    p1 = p1c[:, 0]
    block_expert = bec[:, 0]
    block_valid = bvc[:, 0]

    nw = 32
    dch = 32
    dnch = T // nw // dch
    y = _make_scatter_dispatch(T, P, d, nw, dch)(
        nx2, p0.reshape(nw, dnch, dch), p1.reshape(nw, dnch, dch))

    acc = pl.pallas_call(
        _shared_kernel,
        grid=(nt,),
        in_specs=[
            pl.BlockSpec((TB, d), lambda t: (t, 0)),
            pl.BlockSpec((TB, d), lambda t: (t, 0)),
            pl.BlockSpec((d, dff), lambda t: (0, 0)),
            pl.BlockSpec((d, dff), lambda t: (0, 0)),
            pl.BlockSpec((dff, d), lambda t: (0, 0)),
        ],
        out_specs=pl.BlockSpec((TB, d), lambda t: (t, 0)),
        out_shape=jax.ShapeDtypeStruct((T, d), f32),
    )(x1, nx2, sh_wg, sh_wu, sh_wd)

    z = pl.pallas_call(
        _group_kernel,
        grid_spec=pltpu.PrefetchScalarGridSpec(
            num_scalar_prefetch=2,
            grid=(NB,),
            in_specs=[
                pl.BlockSpec((BLK, d), lambda bb, be, bv: (bb, 0)),
                pl.BlockSpec((1, d, dff), lambda bb, be, bv: (be[bb], 0, 0)),
                pl.BlockSpec((1, d, dff), lambda bb, be, bv: (be[bb], 0, 0)),
                pl.BlockSpec((1, dff, d), lambda bb, be, bv: (be[bb], 0, 0)),
            ],
            out_specs=pl.BlockSpec((BLK, d), lambda bb, be, bv: (bb, 0)),
        ),
        out_shape=jax.ShapeDtypeStruct((P, d), f32),
        compiler_params=pltpu.CompilerParams(
            dimension_semantics=("arbitrary",)),
    )(block_expert, block_valid, y, ex_wg, ex_wu, ex_wd)

    z0, z1 = _make_gather2(T, d, nw, 16, f32)(z, p0, p1)
    out = pl.pallas_call(
        _combine_add_kernel,
        grid=(nt,),
        in_specs=[
            pl.BlockSpec((TB, d), lambda t: (t, 0)),
            pl.BlockSpec((TB, d), lambda t: (t, 0)),
            pl.BlockSpec((TB, d), lambda t: (t, 0)),
            pl.BlockSpec((TB, 1), lambda t: (t, 0)),
            pl.BlockSpec((TB, 1), lambda t: (t, 0)),
        ],
        out_specs=pl.BlockSpec((TB, d), lambda t: (t, 0)),
        out_shape=jax.ShapeDtypeStruct((T, d), f32),
    )(acc, z0, z1, w1, w2)

    return out.reshape(b, s, d)
